# Initial kernel scaffold; baseline (speedup 1.0000x reference)
import jax
import jax.numpy as jnp
from jax.experimental import pallas as pl


def kernel(ufeat, ifeat, Q, edge_index):
    raise NotImplementedError("write your pallas kernel here")


# speedup vs baseline: 1.4290x; 1.4290x over previous
import functools

import jax
import jax.numpy as jnp
from jax import lax
from jax.experimental import pallas as pl
from jax.experimental.pallas import tpu as pltpu
from jax.experimental.pallas import tpu_sc as plsc

N_FEAT = 256
L = 16
NC = 2
NS = 16
NW = NC * NS
C = 128


def _dot_kernel(src_hbm, dst_hbm, ufeat_hbm, ifeat_hbm, out_hbm,
                sidx, didx, urows, vrows, outv, s_u, s_v):
    wid = lax.axis_index("s") * NC + lax.axis_index("c")
    e_per_w = src_hbm.shape[0] // NW
    n_chunks = e_per_w // C
    row_ids = lax.iota(jnp.int32, L)

    def chunk_body(i, _):
        goff = wid * e_per_w + i * C
        pltpu.sync_copy(src_hbm.at[pl.ds(goff, C)], sidx)
        pltpu.sync_copy(dst_hbm.at[pl.ds(goff, C)], didx)
        cu = pltpu.async_copy(ufeat_hbm.at[sidx], urows, s_u)
        cv = pltpu.async_copy(ifeat_hbm.at[didx], vrows, s_v)
        cu.wait()
        cv.wait()

        def group_body(g, _):
            base = g * L
            res = jnp.zeros((L,), jnp.float32)
            for e in range(L):
                acc = urows[base + e, pl.ds(0, L)] * vrows[base + e, pl.ds(0, L)]
                for j in range(1, N_FEAT // L):
                    acc += (urows[base + e, pl.ds(j * L, L)]
                            * vrows[base + e, pl.ds(j * L, L)])
                s = jnp.sum(acc)
                res = jnp.where(row_ids == e, s, res)
            outv[pl.ds(base, L)] = res
            return 0

        lax.fori_loop(0, C // L, group_body, 0)
        pltpu.sync_copy(outv, out_hbm.at[pl.ds(goff, C)])
        return 0

    lax.fori_loop(0, n_chunks, chunk_body, 0)


def kernel(ufeat, ifeat, Q, edge_index):
    del Q
    e = edge_index.shape[1]
    src = edge_index[0].astype(jnp.int32)
    dst = edge_index[1].astype(jnp.int32)
    e_pad = ((e + NW * C - 1) // (NW * C)) * (NW * C)
    if e_pad != e:
        src = jnp.pad(src, (0, e_pad - e))
        dst = jnp.pad(dst, (0, e_pad - e))

    run = pl.kernel(
        _dot_kernel,
        out_type=jax.ShapeDtypeStruct((e_pad,), jnp.float32),
        mesh=plsc.VectorSubcoreMesh(
            core_axis_name="c", subcore_axis_name="s",
            num_cores=NC, num_subcores=NS),
        scratch_types=[
            pltpu.VMEM((C,), jnp.int32),
            pltpu.VMEM((C,), jnp.int32),
            pltpu.VMEM((C, N_FEAT), jnp.float32),
            pltpu.VMEM((C, N_FEAT), jnp.float32),
            pltpu.VMEM((C,), jnp.float32),
            pltpu.SemaphoreType.DMA,
            pltpu.SemaphoreType.DMA,
        ],
        compiler_params=pltpu.CompilerParams(needs_layout_passes=False),
    )
    out = run(src, dst, ufeat, ifeat)
    return out[:e, None]


# speedup vs baseline: 2.3524x; 1.6462x over previous
import functools

import jax
import jax.numpy as jnp
from jax import lax
from jax.experimental import pallas as pl
from jax.experimental.pallas import tpu as pltpu
from jax.experimental.pallas import tpu_sc as plsc

N_FEAT = 256
L = 16
NC = 2
NS = 16
NW = NC * NS
C = 80


def _dot_kernel(src_hbm, dst_hbm, ufeat_hbm, ifeat_hbm, out_hbm,
                sidx, didx, u0, u1, v0, v1, o0, o1,
                su0, su1, sv0, sv1, so0, so1):
    wid = lax.axis_index("s") * NC + lax.axis_index("c")
    e_per_w = src_hbm.shape[0] // NW
    n_chunks = e_per_w // C
    w_base = wid * e_per_w
    row_ids = lax.iota(jnp.int32, L)
    ubuf = (u0, u1)
    vbuf = (v0, v1)
    obuf = (o0, o1)
    usem = (su0, su1)
    vsem = (sv0, sv1)
    osem = (so0, so1)

    pltpu.sync_copy(src_hbm.at[pl.ds(w_base, e_per_w)], sidx)
    pltpu.sync_copy(dst_hbm.at[pl.ds(w_base, e_per_w)], didx)

    def fire(i, b):
        off = i * C
        pltpu.async_copy(ufeat_hbm.at[sidx.at[pl.ds(off, C)]], ubuf[b], usem[b])
        pltpu.async_copy(ifeat_hbm.at[didx.at[pl.ds(off, C)]], vbuf[b], vsem[b])

    def wait_rows(b):
        pltpu.make_async_copy(
            ufeat_hbm.at[sidx.at[pl.ds(0, C)]], ubuf[b], usem[b]).wait()
        pltpu.make_async_copy(
            ifeat_hbm.at[didx.at[pl.ds(0, C)]], vbuf[b], vsem[b]).wait()

    def wait_out(b):
        pltpu.make_async_copy(
            obuf[b], out_hbm.at[pl.ds(0, C)], osem[b]).wait()

    fire(0, 0)

    def outer_body(o, _):
        for b in range(2):
            i = o * 2 + b
            wait_rows(b)

            @pl.when(i + 1 < n_chunks)
            def _():
                fire(i + 1, 1 - b)

            @pl.when(i >= 2)
            def _():
                wait_out(b)

            urows = ubuf[b]
            vrows = vbuf[b]

            def group_body(g, _):
                base = g * L
                res = jnp.zeros((L,), jnp.float32)
                for e in range(L):
                    acc = (urows[base + e, pl.ds(0, L)]
                           * vrows[base + e, pl.ds(0, L)])
                    for j in range(1, N_FEAT // L):
                        acc += (urows[base + e, pl.ds(j * L, L)]
                                * vrows[base + e, pl.ds(j * L, L)])
                    s = jnp.sum(acc)
                    res = jnp.where(row_ids == e, s, res)
                obuf[b][pl.ds(base, L)] = res
                return 0

            lax.fori_loop(0, C // L, group_body, 0)
            pltpu.async_copy(obuf[b], out_hbm.at[pl.ds(w_base + i * C, C)],
                             osem[b])
        return 0

    lax.fori_loop(0, n_chunks // 2, outer_body, 0)
    wait_out(0)
    wait_out(1)


def kernel(ufeat, ifeat, Q, edge_index):
    del Q
    e = edge_index.shape[1]
    src = edge_index[0].astype(jnp.int32)
    dst = edge_index[1].astype(jnp.int32)
    blk = NW * C * 2
    e_pad = ((e + blk - 1) // blk) * blk
    if e_pad != e:
        src = jnp.pad(src, (0, e_pad - e))
        dst = jnp.pad(dst, (0, e_pad - e))
    e_per_w = e_pad // NW

    run = pl.kernel(
        _dot_kernel,
        out_type=jax.ShapeDtypeStruct((e_pad,), jnp.float32),
        mesh=plsc.VectorSubcoreMesh(
            core_axis_name="c", subcore_axis_name="s",
            num_cores=NC, num_subcores=NS),
        scratch_types=[
            pltpu.VMEM((e_per_w,), jnp.int32),
            pltpu.VMEM((e_per_w,), jnp.int32),
            pltpu.VMEM((C, N_FEAT), jnp.float32),
            pltpu.VMEM((C, N_FEAT), jnp.float32),
            pltpu.VMEM((C, N_FEAT), jnp.float32),
            pltpu.VMEM((C, N_FEAT), jnp.float32),
            pltpu.VMEM((C,), jnp.float32),
            pltpu.VMEM((C,), jnp.float32),
            pltpu.SemaphoreType.DMA,
            pltpu.SemaphoreType.DMA,
            pltpu.SemaphoreType.DMA,
            pltpu.SemaphoreType.DMA,
            pltpu.SemaphoreType.DMA,
            pltpu.SemaphoreType.DMA,
        ],
        compiler_params=pltpu.CompilerParams(needs_layout_passes=False),
    )
    out = run(src, dst, ufeat, ifeat)
    return out[:e, None]
